# baseline (speedup 1.0000x reference)
import functools

import jax
import jax.numpy as jnp
from jax import lax
from jax.experimental import pallas as pl
from jax.experimental.pallas import tpu as pltpu
from jax.experimental.pallas import tpu_sc as plsc

F32 = jnp.float32
NC = 2
NS = 16
NW = NC * NS
CH = 64
NBUF = 5


def _sc_deg(rowp, ewp, n_pad):
    nch = rowp.shape[1]
    chd = rowp.shape[2]
    per_sub = n_pad // NS
    mesh = plsc.VectorSubcoreMesh(core_axis_name="c", subcore_axis_name="s")

    @functools.partial(
        pl.kernel,
        mesh=mesh,
        out_type=[
            jax.ShapeDtypeStruct((n_pad,), F32),
            jax.ShapeDtypeStruct((n_pad,), F32),
        ],
        scratch_types=[
            pltpu.VMEM((nch, chd), jnp.int32),
            pltpu.VMEM((nch, chd), F32),
            pltpu.VMEM((per_sub,), F32),
            pltpu.VMEM_SHARED((n_pad,), F32),
            pltpu.SemaphoreType.DMA,
        ],
    )
    def k(row_hbm, ew_hbm, deg0_out, deg1_out, row_v, ew_v, vbuf, dacc, sem):
        cid = lax.axis_index("c")
        sid = lax.axis_index("s")
        wid = cid * NS + sid

        pltpu.async_copy(row_hbm.at[wid], row_v, sem).wait()
        pltpu.async_copy(ew_hbm.at[wid], ew_v, sem).wait()

        def zb(t, _):
            vbuf[pl.ds(t * 16, 16)] = jnp.zeros((16,), F32)
            return _
        lax.fori_loop(0, per_sub // 16, zb, None)
        pltpu.sync_copy(vbuf, dacc.at[pl.ds(sid * per_sub, per_sub)])
        plsc.subcore_barrier()

        def body(j, _):
            pltpu.sync_copy(ew_v.at[j], dacc.at[row_v.at[j]], add=True)
            return _
        lax.fori_loop(0, nch, body, None)
        plsc.subcore_barrier()

        pltpu.sync_copy(dacc.at[pl.ds(sid * per_sub, per_sub)], vbuf)

        @pl.when(cid == 0)
        def _():
            pltpu.sync_copy(vbuf, deg0_out.at[pl.ds(sid * per_sub, per_sub)])

        @pl.when(cid == 1)
        def _():
            pltpu.sync_copy(vbuf, deg1_out.at[pl.ds(sid * per_sub, per_sub)])

    return k(rowp, ewp)


def _zero_rows(buf, rows):
    def zb(r, _):
        for t in range(8):
            buf[r, pl.ds(t * 16, 16)] = jnp.zeros((16,), F32)
        return _
    lax.fori_loop(0, rows, zb, None)


def _scatter_phase(src_hbm, out_hbm, acc, row4, col4, ew4, wid,
                   gb, idxb, ewb, gsem, rsem, csem, esem, ssem,
                   cid, sid, nch, n_acc):
    per_sub = n_acc // NS
    wchunk = 64
    nw = per_sub // wchunk

    _zero_rows(gb[0], wchunk)
    zd = []
    for q in range(nw):
        zd.append(pltpu.async_copy(
            gb[0].at[pl.ds(0, wchunk)],
            acc.at[pl.ds(sid * per_sub + q * wchunk, wchunk)],
            gsem[0]))
    for d_ in zd:
        d_.wait()
    plsc.subcore_barrier()

    for b in range(2):
        pltpu.async_copy(row4.at[wid, b], idxb.at[pl.ds(b, 1)], rsem[b])
        pltpu.async_copy(col4.at[wid, b], idxb.at[pl.ds(NBUF + b, 1)],
                         csem[b])
        pltpu.async_copy(ew4.at[wid, b], ewb.at[pl.ds(b, 1)], esem[b])
    pltpu.make_async_copy(row4.at[wid, 0], idxb.at[pl.ds(0, 1)],
                          rsem[0]).wait()
    pltpu.async_copy(src_hbm.at[idxb.at[0]], gb[0], gsem[0])

    @pl.loop(0, nch, step=NBUF)
    def _loop(j3):
        for b in range(NBUF):
            j = j3 + b
            gbuf = gb[b]
            b1 = (b + 1) % NBUF
            b2 = (b + 2) % NBUF

            pltpu.make_async_copy(src_hbm.at[idxb.at[b]], gbuf,
                                  gsem[b]).wait()
            pltpu.make_async_copy(ew4.at[wid, j], ewb.at[pl.ds(b, 1)],
                                  esem[b]).wait()

            def scale(q, _, gbuf=gbuf, b=b):
                nv16 = ewb[b, pl.ds(q * 16, 16)]
                for u in range(16):
                    nv = nv16[u]
                    e = q * 16 + u
                    for t in range(8):
                        sl = pl.ds(t * 16, 16)
                        gbuf[e, sl] = gbuf[e, sl] * nv
                return _
            lax.fori_loop(0, CH // 16, scale, None)

            pltpu.make_async_copy(col4.at[wid, j],
                                  idxb.at[pl.ds(NBUF + b, 1)],
                                  csem[b]).wait()
            pltpu.async_copy(gbuf, acc.at[idxb.at[NBUF + b]], ssem[b],
                             add=True)

            @pl.when(j + 1 < nch)
            def _():
                pltpu.make_async_copy(
                    row4.at[wid, j], idxb.at[pl.ds(b1, 1)], rsem[b1]).wait()
                pltpu.async_copy(src_hbm.at[idxb.at[b1]], gb[b1], gsem[b1])

            @pl.when(j >= NBUF - 2)
            def _():
                pltpu.make_async_copy(
                    gb[b2], acc.at[idxb.at[NBUF + b2]], ssem[b2]).wait()

            @pl.when(j + 2 < nch)
            def _():
                jn = j + 2
                pltpu.async_copy(row4.at[wid, jn], idxb.at[pl.ds(b2, 1)],
                                 rsem[b2])
                pltpu.async_copy(col4.at[wid, jn],
                                 idxb.at[pl.ds(NBUF + b2, 1)], csem[b2])
                pltpu.async_copy(ew4.at[wid, jn], ewb.at[pl.ds(b2, 1)],
                                 esem[b2])

    for k in range(nch - (NBUF - 2), nch):
        bl = k % NBUF
        pltpu.make_async_copy(gb[bl], acc.at[idxb.at[NBUF + bl]],
                              ssem[bl]).wait()
    plsc.subcore_barrier()

    wdesc = [None, None]
    for q in range(nw):
        b = q % 2
        r0 = sid * per_sub + q * wchunk
        if wdesc[b] is not None:
            wdesc[b].wait()
        pltpu.async_copy(acc.at[pl.ds(r0, wchunk)],
                         gb[b].at[pl.ds(0, wchunk)], gsem[b]).wait()
        wdesc[b] = pltpu.async_copy(
            gb[b].at[pl.ds(0, wchunk)],
            out_hbm.at[cid, pl.ds(r0, wchunk)], ssem[b])
    for b in range(2):
        if wdesc[b] is not None:
            wdesc[b].wait()
    plsc.subcore_barrier()


def _sc_scratch(d):
    return ([pltpu.VMEM((CH, d), F32)] * NBUF
            + [pltpu.VMEM((2 * NBUF, CH), jnp.int32)]
            + [pltpu.VMEM((NBUF, CH), F32)]
            + [pltpu.SemaphoreType.DMA] * (5 * NBUF))


def _sc_scatter_xh(row4, col4, ew4, xd, hd, n_pad):
    nch = row4.shape[1]
    d = xd.shape[1]
    mesh = plsc.VectorSubcoreMesh(core_axis_name="c", subcore_axis_name="s")

    @functools.partial(
        pl.kernel,
        mesh=mesh,
        out_type=[
            jax.ShapeDtypeStruct((NC, n_pad, d), F32),
            jax.ShapeDtypeStruct((NC, n_pad, d), F32),
        ],
        scratch_types=_sc_scratch(d) + [
            pltpu.VMEM_SHARED((n_pad, d), F32),
        ],
    )
    def k(row_hbm, col_hbm, ew_hbm, x_hbm, h_hbm, sx_out, sh_out, *sc):
        cid = lax.axis_index("c")
        sid = lax.axis_index("s")
        wid = cid * NS + sid

        nb = NBUF
        gb = sc[0:nb]
        idxb, ewb = sc[nb], sc[nb + 1]
        base = nb + 2
        gsem, rsem, csem, esem, ssem = (
            sc[base:base + nb], sc[base + nb:base + 2 * nb],
            sc[base + 2 * nb:base + 3 * nb],
            sc[base + 3 * nb:base + 4 * nb],
            sc[base + 4 * nb:base + 5 * nb])
        acc = sc[base + 5 * nb]
        _scatter_phase(x_hbm, sx_out, acc, row_hbm, col_hbm, ew_hbm, wid,
                       gb, idxb, ewb, gsem, rsem, csem, esem, ssem,
                       cid, sid, nch, n_pad)
        _scatter_phase(h_hbm, sh_out, acc, row_hbm, col_hbm, ew_hbm, wid,
                       gb, idxb, ewb, gsem, rsem, csem, esem, ssem,
                       cid, sid, nch, n_pad)

    return k(row4, col4, ew4, xd, hd)


def _sc_scatter1(row4, col4, ew4, src, n_pad):
    nch = row4.shape[1]
    d = src.shape[1]
    mesh = plsc.VectorSubcoreMesh(core_axis_name="c", subcore_axis_name="s")

    @functools.partial(
        pl.kernel,
        mesh=mesh,
        out_type=jax.ShapeDtypeStruct((NC, n_pad, d), F32),
        scratch_types=_sc_scratch(d) + [
            pltpu.VMEM_SHARED((n_pad, d), F32),
        ],
    )
    def k(row_hbm, col_hbm, ew_hbm, src_hbm, s_out, *sc):
        cid = lax.axis_index("c")
        sid = lax.axis_index("s")
        wid = cid * NS + sid

        nb = NBUF
        gb = sc[0:nb]
        idxb, ewb = sc[nb], sc[nb + 1]
        base = nb + 2
        gsem, rsem, csem, esem, ssem = (
            sc[base:base + nb], sc[base + nb:base + 2 * nb],
            sc[base + 2 * nb:base + 3 * nb],
            sc[base + 3 * nb:base + 4 * nb],
            sc[base + 4 * nb:base + 5 * nb])
        acc = sc[base + 5 * nb]
        _scatter_phase(src_hbm, s_out, acc, row_hbm, col_hbm, ew_hbm, wid,
                       gb, idxb, ewb, gsem, rsem, csem, esem, ssem,
                       cid, sid, nch, n_pad)

    return k(row4, col4, ew4, src)


def _tc_prescale(deg0, deg1, x, h, blk):
    n, d = x.shape

    def body(d0, d1, x_r, h_r, dis_o, xd_o, hd_o):
        dg = d0[...] + d1[...]
        pos = dg > 0.0
        dis = jnp.where(pos, lax.rsqrt(jnp.where(pos, dg, 1.0)), 0.0)
        dis_o[...] = dis
        xd_o[...] = x_r[...] * dis
        hd_o[...] = h_r[...] * dis

    return pl.pallas_call(
        body,
        grid=(n // blk,),
        in_specs=[
            pl.BlockSpec((blk, 1), lambda i: (i, 0)),
            pl.BlockSpec((blk, 1), lambda i: (i, 0)),
            pl.BlockSpec((blk, d), lambda i: (i, 0)),
            pl.BlockSpec((blk, d), lambda i: (i, 0)),
        ],
        out_specs=[
            pl.BlockSpec((blk, 1), lambda i: (i, 0)),
            pl.BlockSpec((blk, d), lambda i: (i, 0)),
            pl.BlockSpec((blk, d), lambda i: (i, 0)),
        ],
        out_shape=[
            jax.ShapeDtypeStruct((n, 1), F32),
            jax.ShapeDtypeStruct((n, d), F32),
            jax.ShapeDtypeStruct((n, d), F32),
        ],
    )(deg0, deg1, x, h)


def _tc_gates(x, h, sxp, shp, dis, wzr, bzr, wp, bp, blk):
    n, d = x.shape

    def body(x_r, h_r, sx_r, sh_r, dis_r, wzr_r, bzr_r, wp_r, bp_r,
             z_o, hr_o, hrd_o, p_o):
        nd = -dis_r[...]
        sx = (sx_r[0] + sx_r[1]) * nd
        sh = (sh_r[0] + sh_r[1]) * nd
        u = jnp.concatenate([x_r[...], sx, h_r[...], sh], axis=1)
        g = jnp.dot(u, wzr_r[...], preferred_element_type=F32,
                    precision=lax.Precision.HIGHEST) + bzr_r[...]
        zr = jax.nn.sigmoid(g)
        z_o[...] = zr[:, :d]
        r = zr[:, d:]
        p_o[...] = jnp.dot(u[:, :2 * d], wp_r[...], preferred_element_type=F32,
                           precision=lax.Precision.HIGHEST) + bp_r[...]
        hr = h_r[...] * r
        hr_o[...] = hr
        hrd_o[...] = hr * dis_r[...]

    return pl.pallas_call(
        body,
        grid=(n // blk,),
        in_specs=[
            pl.BlockSpec((blk, d), lambda i: (i, 0)),
            pl.BlockSpec((blk, d), lambda i: (i, 0)),
            pl.BlockSpec((NC, blk, d), lambda i: (0, i, 0)),
            pl.BlockSpec((NC, blk, d), lambda i: (0, i, 0)),
            pl.BlockSpec((blk, 1), lambda i: (i, 0)),
            pl.BlockSpec((4 * d, 2 * d), lambda i: (0, 0)),
            pl.BlockSpec((1, 2 * d), lambda i: (0, 0)),
            pl.BlockSpec((2 * d, d), lambda i: (0, 0)),
            pl.BlockSpec((1, d), lambda i: (0, 0)),
        ],
        out_specs=[
            pl.BlockSpec((blk, d), lambda i: (i, 0)),
            pl.BlockSpec((blk, d), lambda i: (i, 0)),
            pl.BlockSpec((blk, d), lambda i: (i, 0)),
            pl.BlockSpec((blk, d), lambda i: (i, 0)),
        ],
        out_shape=[
            jax.ShapeDtypeStruct((n, d), F32),
            jax.ShapeDtypeStruct((n, d), F32),
            jax.ShapeDtypeStruct((n, d), F32),
            jax.ShapeDtypeStruct((n, d), F32),
        ],
    )(x, h, sxp, shp, dis, wzr, bzr, wp, bp)


def _tc_final(z, hr, p, h, shrp, dis, whh, wenc, benc, blk):
    n, d = z.shape

    def body(z_r, hr_r, p_r, h_r, shr_r, dis_r, whh_r, wenc_r, benc_r,
             h0_o, pool_o, acc):
        i = pl.program_id(0)
        shr = (shr_r[0] + shr_r[1]) * (-dis_r[...])
        v = jnp.concatenate([hr_r[...], shr], axis=1)
        ht = jnp.tanh(p_r[...] + jnp.dot(v, whh_r[...],
                                         preferred_element_type=F32,
                                         precision=lax.Precision.HIGHEST))
        zg = z_r[...]
        h0 = zg * h_r[...] + (1.0 - zg) * ht
        h0_o[...] = h0
        y = jnp.dot(jax.nn.relu(h0), wenc_r[...], preferred_element_type=F32,
                    precision=lax.Precision.HIGHEST) + benc_r[...]
        y = jax.nn.relu(y)

        @pl.when(i == 0)
        def _():
            acc[...] = jnp.zeros_like(acc)

        acc[...] += jnp.sum(y, axis=0, keepdims=True)

        @pl.when(i == pl.num_programs(0) - 1)
        def _():
            pool_o[...] = jax.nn.sigmoid(acc[...] / n)

    return pl.pallas_call(
        body,
        grid=(n // blk,),
        in_specs=[
            pl.BlockSpec((blk, d), lambda i: (i, 0)),
            pl.BlockSpec((blk, d), lambda i: (i, 0)),
            pl.BlockSpec((blk, d), lambda i: (i, 0)),
            pl.BlockSpec((blk, d), lambda i: (i, 0)),
            pl.BlockSpec((NC, blk, d), lambda i: (0, i, 0)),
            pl.BlockSpec((blk, 1), lambda i: (i, 0)),
            pl.BlockSpec((2 * d, d), lambda i: (0, 0)),
            pl.BlockSpec((d, d), lambda i: (0, 0)),
            pl.BlockSpec((1, d), lambda i: (0, 0)),
        ],
        out_specs=[
            pl.BlockSpec((blk, d), lambda i: (i, 0)),
            pl.BlockSpec((1, d), lambda i: (0, 0)),
        ],
        out_shape=[
            jax.ShapeDtypeStruct((n, d), F32),
            jax.ShapeDtypeStruct((1, d), F32),
        ],
        scratch_shapes=[pltpu.VMEM((1, d), F32)],
    )(z, hr, p, h, shrp, dis, whh, wenc, benc)


def kernel(x, edge_index, edge_weight, h, Wxz, bxz, Whz, bhz, Wxr, bxr,
           Whr, bhr, Wxh, bxh, Whh, bhh, Wenc, benc, Wdisc, bdisc):
    n, d = x.shape
    e = edge_index.shape[1]
    del Wdisc, bdisc

    epw = -(-e // NW)
    nch = -(-(-(-epw // CH)) // NBUF) * NBUF
    e_pad = NW * nch * CH
    n_pad = -(-n // (16 * NS)) * (16 * NS)

    row = edge_index[0]
    col = edge_index[1]
    pad = e_pad - e
    pad_idx = (jnp.arange(pad, dtype=jnp.int32) * 37) % n
    row4 = jnp.concatenate([row, pad_idx]).reshape(NW, nch, 1, CH)
    col4 = jnp.concatenate([col, pad_idx]).reshape(NW, nch, 1, CH)
    ew4 = jnp.pad(edge_weight, (0, pad)).reshape(NW, nch, 1, CH)

    chd = 128
    nchd = -(-epw // chd)
    pad_d = NW * nchd * chd - e
    pad_idx_d = (jnp.arange(pad_d, dtype=jnp.int32) * 37) % n
    rowd = jnp.concatenate([row, pad_idx_d]).reshape(NW, nchd, chd)
    ewd = jnp.pad(edge_weight, (0, pad_d)).reshape(NW, nchd, chd)

    deg0, deg1 = _sc_deg(rowd, ewd, n_pad)
    blk = 1000
    dis, xd, hd = _tc_prescale(deg0[:n].reshape(n, 1), deg1[:n].reshape(n, 1),
                               x, h, blk)

    sxp, shp = _sc_scatter_xh(row4, col4, ew4, xd, hd, n_pad)

    wz = jnp.concatenate([Wxz[0], Wxz[1], Whz[0], Whz[1]], axis=0)
    wr = jnp.concatenate([Wxr[0], Wxr[1], Whr[0], Whr[1]], axis=0)
    wzr = jnp.concatenate([wz, wr], axis=1)
    bzr = jnp.concatenate([bxz + bhz, bxr + bhr]).reshape(1, 2 * d)
    wp = jnp.concatenate([Wxh[0], Wxh[1]], axis=0)
    bp = (bxh + bhh).reshape(1, d)

    z, hr, hrd, p = _tc_gates(x, h, sxp, shp, dis, wzr, bzr, wp, bp, blk)

    shrp = _sc_scatter1(row4, col4, ew4, hrd, n_pad)

    whh = jnp.concatenate([Whh[0], Whh[1]], axis=0)
    h0, pool = _tc_final(z, hr, p, h, shrp, dis, whh, Wenc,
                         benc.reshape(1, -1), blk)
    return (pool.reshape(-1), h0)


# speedup vs baseline: 1.0964x; 1.0964x over previous
import functools

import jax
import jax.numpy as jnp
from jax import lax
from jax.experimental import pallas as pl
from jax.experimental.pallas import tpu as pltpu
from jax.experimental.pallas import tpu_sc as plsc

F32 = jnp.float32
NC = 2
NS = 16
NW = NC * NS
CH = 80
NBUF = 4


def _sc_deg(rowp, ewp, n_pad):
    nch = rowp.shape[1]
    chd = rowp.shape[2]
    per_sub = n_pad // NS
    mesh = plsc.VectorSubcoreMesh(core_axis_name="c", subcore_axis_name="s")

    @functools.partial(
        pl.kernel,
        mesh=mesh,
        out_type=[
            jax.ShapeDtypeStruct((n_pad,), F32),
            jax.ShapeDtypeStruct((n_pad,), F32),
        ],
        scratch_types=[
            pltpu.VMEM((nch, chd), jnp.int32),
            pltpu.VMEM((nch, chd), F32),
            pltpu.VMEM((per_sub,), F32),
            pltpu.VMEM_SHARED((n_pad,), F32),
            pltpu.SemaphoreType.DMA,
        ],
    )
    def k(row_hbm, ew_hbm, deg0_out, deg1_out, row_v, ew_v, vbuf, dacc, sem):
        cid = lax.axis_index("c")
        sid = lax.axis_index("s")
        wid = cid * NS + sid

        pltpu.async_copy(row_hbm.at[wid], row_v, sem).wait()
        pltpu.async_copy(ew_hbm.at[wid], ew_v, sem).wait()

        def zb(t, _):
            vbuf[pl.ds(t * 16, 16)] = jnp.zeros((16,), F32)
            return _
        lax.fori_loop(0, per_sub // 16, zb, None)
        pltpu.sync_copy(vbuf, dacc.at[pl.ds(sid * per_sub, per_sub)])
        plsc.subcore_barrier()

        def body(j, _):
            pltpu.sync_copy(ew_v.at[j], dacc.at[row_v.at[j]], add=True)
            return _
        lax.fori_loop(0, nch, body, None)
        plsc.subcore_barrier()

        pltpu.sync_copy(dacc.at[pl.ds(sid * per_sub, per_sub)], vbuf)

        @pl.when(cid == 0)
        def _():
            pltpu.sync_copy(vbuf, deg0_out.at[pl.ds(sid * per_sub, per_sub)])

        @pl.when(cid == 1)
        def _():
            pltpu.sync_copy(vbuf, deg1_out.at[pl.ds(sid * per_sub, per_sub)])

    return k(rowp, ewp)


def _zero_rows(buf, rows):
    def zb(r, _):
        for t in range(8):
            buf[r, pl.ds(t * 16, 16)] = jnp.zeros((16,), F32)
        return _
    lax.fori_loop(0, rows, zb, None)


def _scatter_phase(src_hbm, out_hbm, acc, row4, col4, ew4, wid,
                   gb, idxb, ewb, gsem, rsem, csem, esem, ssem,
                   cid, sid, nch, n_acc):
    per_sub = n_acc // NS
    wchunk = 64
    nw = per_sub // wchunk

    _zero_rows(gb[0], wchunk)
    zd = []
    for q in range(nw):
        zd.append(pltpu.async_copy(
            gb[0].at[pl.ds(0, wchunk)],
            acc.at[pl.ds(sid * per_sub + q * wchunk, wchunk)],
            gsem[0]))
    for d_ in zd:
        d_.wait()
    plsc.subcore_barrier()

    for b in range(2):
        pltpu.async_copy(row4.at[wid, b], idxb.at[pl.ds(b, 1)], rsem[b])
        pltpu.async_copy(col4.at[wid, b], idxb.at[pl.ds(NBUF + b, 1)],
                         csem[b])
        pltpu.async_copy(ew4.at[wid, b], ewb.at[pl.ds(b, 1)], esem[b])
    pltpu.make_async_copy(row4.at[wid, 0], idxb.at[pl.ds(0, 1)],
                          rsem[0]).wait()
    pltpu.async_copy(src_hbm.at[idxb.at[0]], gb[0], gsem[0])

    @pl.loop(0, nch, step=NBUF)
    def _loop(j3):
        for b in range(NBUF):
            j = j3 + b
            gbuf = gb[b]
            b1 = (b + 1) % NBUF
            b2 = (b + 2) % NBUF

            pltpu.make_async_copy(src_hbm.at[idxb.at[b]], gbuf,
                                  gsem[b]).wait()
            pltpu.make_async_copy(ew4.at[wid, j], ewb.at[pl.ds(b, 1)],
                                  esem[b]).wait()

            def scale(q, _, gbuf=gbuf, b=b):
                nv16 = ewb[b, pl.ds(q * 16, 16)]
                for u in range(16):
                    nv = nv16[u]
                    e = q * 16 + u
                    for t in range(8):
                        sl = pl.ds(t * 16, 16)
                        gbuf[e, sl] = gbuf[e, sl] * nv
                return _
            lax.fori_loop(0, CH // 16, scale, None)

            pltpu.make_async_copy(col4.at[wid, j],
                                  idxb.at[pl.ds(NBUF + b, 1)],
                                  csem[b]).wait()
            pltpu.async_copy(gbuf, acc.at[idxb.at[NBUF + b]], ssem[b],
                             add=True)

            @pl.when(j + 1 < nch)
            def _():
                pltpu.make_async_copy(
                    row4.at[wid, j], idxb.at[pl.ds(b1, 1)], rsem[b1]).wait()
                pltpu.async_copy(src_hbm.at[idxb.at[b1]], gb[b1], gsem[b1])

            @pl.when(j >= NBUF - 2)
            def _():
                pltpu.make_async_copy(
                    gb[b2], acc.at[idxb.at[NBUF + b2]], ssem[b2]).wait()

            @pl.when(j + 2 < nch)
            def _():
                jn = j + 2
                pltpu.async_copy(row4.at[wid, jn], idxb.at[pl.ds(b2, 1)],
                                 rsem[b2])
                pltpu.async_copy(col4.at[wid, jn],
                                 idxb.at[pl.ds(NBUF + b2, 1)], csem[b2])
                pltpu.async_copy(ew4.at[wid, jn], ewb.at[pl.ds(b2, 1)],
                                 esem[b2])

    for k in range(nch - (NBUF - 2), nch):
        bl = k % NBUF
        pltpu.make_async_copy(gb[bl], acc.at[idxb.at[NBUF + bl]],
                              ssem[bl]).wait()
    plsc.subcore_barrier()

    wdesc = [None, None]
    for q in range(nw):
        b = q % 2
        r0 = sid * per_sub + q * wchunk
        if wdesc[b] is not None:
            wdesc[b].wait()
        pltpu.async_copy(acc.at[pl.ds(r0, wchunk)],
                         gb[b].at[pl.ds(0, wchunk)], gsem[b]).wait()
        wdesc[b] = pltpu.async_copy(
            gb[b].at[pl.ds(0, wchunk)],
            out_hbm.at[cid, pl.ds(r0, wchunk)], ssem[b])
    for b in range(2):
        if wdesc[b] is not None:
            wdesc[b].wait()
    plsc.subcore_barrier()


def _sc_scratch(d):
    return ([pltpu.VMEM((CH, d), F32)] * NBUF
            + [pltpu.VMEM((2 * NBUF, CH), jnp.int32)]
            + [pltpu.VMEM((NBUF, CH), F32)]
            + [pltpu.SemaphoreType.DMA] * (5 * NBUF))


def _sc_scatter_xh(row4, col4, ew4, xd, hd, n_pad):
    nch = row4.shape[1]
    d = xd.shape[1]
    mesh = plsc.VectorSubcoreMesh(core_axis_name="c", subcore_axis_name="s")

    @functools.partial(
        pl.kernel,
        mesh=mesh,
        out_type=[
            jax.ShapeDtypeStruct((NC, n_pad, d), F32),
            jax.ShapeDtypeStruct((NC, n_pad, d), F32),
        ],
        scratch_types=_sc_scratch(d) + [
            pltpu.VMEM_SHARED((n_pad, d), F32),
        ],
    )
    def k(row_hbm, col_hbm, ew_hbm, x_hbm, h_hbm, sx_out, sh_out, *sc):
        cid = lax.axis_index("c")
        sid = lax.axis_index("s")
        wid = cid * NS + sid

        nb = NBUF
        gb = sc[0:nb]
        idxb, ewb = sc[nb], sc[nb + 1]
        base = nb + 2
        gsem, rsem, csem, esem, ssem = (
            sc[base:base + nb], sc[base + nb:base + 2 * nb],
            sc[base + 2 * nb:base + 3 * nb],
            sc[base + 3 * nb:base + 4 * nb],
            sc[base + 4 * nb:base + 5 * nb])
        acc = sc[base + 5 * nb]
        _scatter_phase(x_hbm, sx_out, acc, row_hbm, col_hbm, ew_hbm, wid,
                       gb, idxb, ewb, gsem, rsem, csem, esem, ssem,
                       cid, sid, nch, n_pad)
        _scatter_phase(h_hbm, sh_out, acc, row_hbm, col_hbm, ew_hbm, wid,
                       gb, idxb, ewb, gsem, rsem, csem, esem, ssem,
                       cid, sid, nch, n_pad)

    return k(row4, col4, ew4, xd, hd)


def _sc_scatter1(row4, col4, ew4, src, n_pad):
    nch = row4.shape[1]
    d = src.shape[1]
    mesh = plsc.VectorSubcoreMesh(core_axis_name="c", subcore_axis_name="s")

    @functools.partial(
        pl.kernel,
        mesh=mesh,
        out_type=jax.ShapeDtypeStruct((NC, n_pad, d), F32),
        scratch_types=_sc_scratch(d) + [
            pltpu.VMEM_SHARED((n_pad, d), F32),
        ],
    )
    def k(row_hbm, col_hbm, ew_hbm, src_hbm, s_out, *sc):
        cid = lax.axis_index("c")
        sid = lax.axis_index("s")
        wid = cid * NS + sid

        nb = NBUF
        gb = sc[0:nb]
        idxb, ewb = sc[nb], sc[nb + 1]
        base = nb + 2
        gsem, rsem, csem, esem, ssem = (
            sc[base:base + nb], sc[base + nb:base + 2 * nb],
            sc[base + 2 * nb:base + 3 * nb],
            sc[base + 3 * nb:base + 4 * nb],
            sc[base + 4 * nb:base + 5 * nb])
        acc = sc[base + 5 * nb]
        _scatter_phase(src_hbm, s_out, acc, row_hbm, col_hbm, ew_hbm, wid,
                       gb, idxb, ewb, gsem, rsem, csem, esem, ssem,
                       cid, sid, nch, n_pad)

    return k(row4, col4, ew4, src)


def _tc_prescale(deg0, deg1, x, h, blk):
    n, d = x.shape

    def body(d0, d1, x_r, h_r, dis_o, xd_o, hd_o):
        dg = d0[...] + d1[...]
        pos = dg > 0.0
        dis = jnp.where(pos, lax.rsqrt(jnp.where(pos, dg, 1.0)), 0.0)
        dis_o[...] = dis
        xd_o[...] = x_r[...] * dis
        hd_o[...] = h_r[...] * dis

    return pl.pallas_call(
        body,
        grid=(n // blk,),
        in_specs=[
            pl.BlockSpec((blk, 1), lambda i: (i, 0)),
            pl.BlockSpec((blk, 1), lambda i: (i, 0)),
            pl.BlockSpec((blk, d), lambda i: (i, 0)),
            pl.BlockSpec((blk, d), lambda i: (i, 0)),
        ],
        out_specs=[
            pl.BlockSpec((blk, 1), lambda i: (i, 0)),
            pl.BlockSpec((blk, d), lambda i: (i, 0)),
            pl.BlockSpec((blk, d), lambda i: (i, 0)),
        ],
        out_shape=[
            jax.ShapeDtypeStruct((n, 1), F32),
            jax.ShapeDtypeStruct((n, d), F32),
            jax.ShapeDtypeStruct((n, d), F32),
        ],
    )(deg0, deg1, x, h)


def _tc_gates(x, h, sxp, shp, dis, wzr, bzr, wp, bp, blk):
    n, d = x.shape

    def body(x_r, h_r, sx_r, sh_r, dis_r, wzr_r, bzr_r, wp_r, bp_r,
             z_o, hr_o, hrd_o, p_o):
        nd = -dis_r[...]
        sx = (sx_r[0] + sx_r[1]) * nd
        sh = (sh_r[0] + sh_r[1]) * nd
        u = jnp.concatenate([x_r[...], sx, h_r[...], sh], axis=1)
        g = jnp.dot(u, wzr_r[...], preferred_element_type=F32,
                    precision=lax.Precision.HIGHEST) + bzr_r[...]
        zr = jax.nn.sigmoid(g)
        z_o[...] = zr[:, :d]
        r = zr[:, d:]
        p_o[...] = jnp.dot(u[:, :2 * d], wp_r[...], preferred_element_type=F32,
                           precision=lax.Precision.HIGHEST) + bp_r[...]
        hr = h_r[...] * r
        hr_o[...] = hr
        hrd_o[...] = hr * dis_r[...]

    return pl.pallas_call(
        body,
        grid=(n // blk,),
        in_specs=[
            pl.BlockSpec((blk, d), lambda i: (i, 0)),
            pl.BlockSpec((blk, d), lambda i: (i, 0)),
            pl.BlockSpec((NC, blk, d), lambda i: (0, i, 0)),
            pl.BlockSpec((NC, blk, d), lambda i: (0, i, 0)),
            pl.BlockSpec((blk, 1), lambda i: (i, 0)),
            pl.BlockSpec((4 * d, 2 * d), lambda i: (0, 0)),
            pl.BlockSpec((1, 2 * d), lambda i: (0, 0)),
            pl.BlockSpec((2 * d, d), lambda i: (0, 0)),
            pl.BlockSpec((1, d), lambda i: (0, 0)),
        ],
        out_specs=[
            pl.BlockSpec((blk, d), lambda i: (i, 0)),
            pl.BlockSpec((blk, d), lambda i: (i, 0)),
            pl.BlockSpec((blk, d), lambda i: (i, 0)),
            pl.BlockSpec((blk, d), lambda i: (i, 0)),
        ],
        out_shape=[
            jax.ShapeDtypeStruct((n, d), F32),
            jax.ShapeDtypeStruct((n, d), F32),
            jax.ShapeDtypeStruct((n, d), F32),
            jax.ShapeDtypeStruct((n, d), F32),
        ],
    )(x, h, sxp, shp, dis, wzr, bzr, wp, bp)


def _tc_final(z, hr, p, h, shrp, dis, whh, wenc, benc, blk):
    n, d = z.shape

    def body(z_r, hr_r, p_r, h_r, shr_r, dis_r, whh_r, wenc_r, benc_r,
             h0_o, pool_o, acc):
        i = pl.program_id(0)
        shr = (shr_r[0] + shr_r[1]) * (-dis_r[...])
        v = jnp.concatenate([hr_r[...], shr], axis=1)
        ht = jnp.tanh(p_r[...] + jnp.dot(v, whh_r[...],
                                         preferred_element_type=F32,
                                         precision=lax.Precision.HIGHEST))
        zg = z_r[...]
        h0 = zg * h_r[...] + (1.0 - zg) * ht
        h0_o[...] = h0
        y = jnp.dot(jax.nn.relu(h0), wenc_r[...], preferred_element_type=F32,
                    precision=lax.Precision.HIGHEST) + benc_r[...]
        y = jax.nn.relu(y)

        @pl.when(i == 0)
        def _():
            acc[...] = jnp.zeros_like(acc)

        acc[...] += jnp.sum(y, axis=0, keepdims=True)

        @pl.when(i == pl.num_programs(0) - 1)
        def _():
            pool_o[...] = jax.nn.sigmoid(acc[...] / n)

    return pl.pallas_call(
        body,
        grid=(n // blk,),
        in_specs=[
            pl.BlockSpec((blk, d), lambda i: (i, 0)),
            pl.BlockSpec((blk, d), lambda i: (i, 0)),
            pl.BlockSpec((blk, d), lambda i: (i, 0)),
            pl.BlockSpec((blk, d), lambda i: (i, 0)),
            pl.BlockSpec((NC, blk, d), lambda i: (0, i, 0)),
            pl.BlockSpec((blk, 1), lambda i: (i, 0)),
            pl.BlockSpec((2 * d, d), lambda i: (0, 0)),
            pl.BlockSpec((d, d), lambda i: (0, 0)),
            pl.BlockSpec((1, d), lambda i: (0, 0)),
        ],
        out_specs=[
            pl.BlockSpec((blk, d), lambda i: (i, 0)),
            pl.BlockSpec((1, d), lambda i: (0, 0)),
        ],
        out_shape=[
            jax.ShapeDtypeStruct((n, d), F32),
            jax.ShapeDtypeStruct((1, d), F32),
        ],
        scratch_shapes=[pltpu.VMEM((1, d), F32)],
    )(z, hr, p, h, shrp, dis, whh, wenc, benc)


def kernel(x, edge_index, edge_weight, h, Wxz, bxz, Whz, bhz, Wxr, bxr,
           Whr, bhr, Wxh, bxh, Whh, bhh, Wenc, benc, Wdisc, bdisc):
    n, d = x.shape
    e = edge_index.shape[1]
    del Wdisc, bdisc

    epw = -(-e // NW)
    nch = -(-(-(-epw // CH)) // NBUF) * NBUF
    e_pad = NW * nch * CH
    n_pad = -(-n // (16 * NS)) * (16 * NS)

    row = edge_index[0]
    col = edge_index[1]
    pad = e_pad - e
    pad_idx = (jnp.arange(pad, dtype=jnp.int32) * 37) % n
    row4 = jnp.concatenate([row, pad_idx]).reshape(NW, nch, 1, CH)
    col4 = jnp.concatenate([col, pad_idx]).reshape(NW, nch, 1, CH)
    ew4 = jnp.pad(edge_weight, (0, pad)).reshape(NW, nch, 1, CH)

    chd = 128
    nchd = -(-epw // chd)
    pad_d = NW * nchd * chd - e
    pad_idx_d = (jnp.arange(pad_d, dtype=jnp.int32) * 37) % n
    rowd = jnp.concatenate([row, pad_idx_d]).reshape(NW, nchd, chd)
    ewd = jnp.pad(edge_weight, (0, pad_d)).reshape(NW, nchd, chd)

    deg0, deg1 = _sc_deg(rowd, ewd, n_pad)
    blk = 1000
    dis, xd, hd = _tc_prescale(deg0[:n].reshape(n, 1), deg1[:n].reshape(n, 1),
                               x, h, blk)

    sxp, shp = _sc_scatter_xh(row4, col4, ew4, xd, hd, n_pad)

    wz = jnp.concatenate([Wxz[0], Wxz[1], Whz[0], Whz[1]], axis=0)
    wr = jnp.concatenate([Wxr[0], Wxr[1], Whr[0], Whr[1]], axis=0)
    wzr = jnp.concatenate([wz, wr], axis=1)
    bzr = jnp.concatenate([bxz + bhz, bxr + bhr]).reshape(1, 2 * d)
    wp = jnp.concatenate([Wxh[0], Wxh[1]], axis=0)
    bp = (bxh + bhh).reshape(1, d)

    z, hr, hrd, p = _tc_gates(x, h, sxp, shp, dis, wzr, bzr, wp, bp, blk)

    shrp = _sc_scatter1(row4, col4, ew4, hrd, n_pad)

    whh = jnp.concatenate([Whh[0], Whh[1]], axis=0)
    h0, pool = _tc_final(z, hr, p, h, shrp, dis, whh, Wenc,
                         benc.reshape(1, -1), blk)
    return (pool.reshape(-1), h0)


# speedup vs baseline: 1.1056x; 1.0084x over previous
import functools

import jax
import jax.numpy as jnp
from jax import lax
from jax.experimental import pallas as pl
from jax.experimental.pallas import tpu as pltpu
from jax.experimental.pallas import tpu_sc as plsc

F32 = jnp.float32
NC = 2
NS = 16
NW = NC * NS
CH = 80
NBUF = 4


def _sc_deg(rowp, ewp, n_pad):
    nch = rowp.shape[1]
    chd = rowp.shape[2]
    per_sub = n_pad // NS
    mesh = plsc.VectorSubcoreMesh(core_axis_name="c", subcore_axis_name="s")

    @functools.partial(
        pl.kernel,
        mesh=mesh,
        out_type=[
            jax.ShapeDtypeStruct((n_pad,), F32),
            jax.ShapeDtypeStruct((n_pad,), F32),
        ],
        scratch_types=[
            pltpu.VMEM((nch, chd), jnp.int32),
            pltpu.VMEM((nch, chd), F32),
            pltpu.VMEM((per_sub,), F32),
            pltpu.VMEM_SHARED((n_pad,), F32),
            pltpu.SemaphoreType.DMA,
        ],
    )
    def k(row_hbm, ew_hbm, deg0_out, deg1_out, row_v, ew_v, vbuf, dacc, sem):
        cid = lax.axis_index("c")
        sid = lax.axis_index("s")
        wid = cid * NS + sid

        pltpu.async_copy(row_hbm.at[wid], row_v, sem).wait()
        pltpu.async_copy(ew_hbm.at[wid], ew_v, sem).wait()

        def zb(t, _):
            vbuf[pl.ds(t * 16, 16)] = jnp.zeros((16,), F32)
            return _
        lax.fori_loop(0, per_sub // 16, zb, None)
        pltpu.sync_copy(vbuf, dacc.at[pl.ds(sid * per_sub, per_sub)])
        plsc.subcore_barrier()

        for j in range(nch):
            pltpu.async_copy(ew_v.at[j], dacc.at[row_v.at[j]], sem, add=True)
            if j >= 8:
                pltpu.make_async_copy(ew_v.at[0], dacc.at[row_v.at[0]],
                                      sem).wait()
        for _ in range(min(nch, 8)):
            pltpu.make_async_copy(ew_v.at[0], dacc.at[row_v.at[0]],
                                  sem).wait()
        plsc.subcore_barrier()

        pltpu.sync_copy(dacc.at[pl.ds(sid * per_sub, per_sub)], vbuf)

        @pl.when(cid == 0)
        def _():
            pltpu.sync_copy(vbuf, deg0_out.at[pl.ds(sid * per_sub, per_sub)])

        @pl.when(cid == 1)
        def _():
            pltpu.sync_copy(vbuf, deg1_out.at[pl.ds(sid * per_sub, per_sub)])

    return k(rowp, ewp)


def _zero_rows(buf, rows):
    def zb(r, _):
        for t in range(8):
            buf[r, pl.ds(t * 16, 16)] = jnp.zeros((16,), F32)
        return _
    lax.fori_loop(0, rows, zb, None)


def _scatter_phase(src_hbm, out_hbm, acc, row4, col4, ew4, wid,
                   gb, idxb, ewb, gsem, rsem, csem, esem, ssem,
                   cid, sid, nch, n_acc):
    per_sub = n_acc // NS
    wchunk = 64
    nw = per_sub // wchunk

    _zero_rows(gb[0], wchunk)
    zd = []
    for q in range(nw):
        zd.append(pltpu.async_copy(
            gb[0].at[pl.ds(0, wchunk)],
            acc.at[pl.ds(sid * per_sub + q * wchunk, wchunk)],
            gsem[0]))
    for d_ in zd:
        d_.wait()
    plsc.subcore_barrier()

    for b in range(2):
        pltpu.async_copy(row4.at[wid, b], idxb.at[pl.ds(b, 1)], rsem[b])
        pltpu.async_copy(col4.at[wid, b], idxb.at[pl.ds(NBUF + b, 1)],
                         csem[b])
        pltpu.async_copy(ew4.at[wid, b], ewb.at[pl.ds(b, 1)], esem[b])
    pltpu.make_async_copy(row4.at[wid, 0], idxb.at[pl.ds(0, 1)],
                          rsem[0]).wait()
    pltpu.async_copy(src_hbm.at[idxb.at[0]], gb[0], gsem[0])

    @pl.loop(0, nch, step=NBUF)
    def _loop(j3):
        for b in range(NBUF):
            j = j3 + b
            gbuf = gb[b]
            b1 = (b + 1) % NBUF
            b2 = (b + 2) % NBUF

            pltpu.make_async_copy(src_hbm.at[idxb.at[b]], gbuf,
                                  gsem[b]).wait()
            pltpu.make_async_copy(ew4.at[wid, j], ewb.at[pl.ds(b, 1)],
                                  esem[b]).wait()

            def scale(q, _, gbuf=gbuf, b=b):
                nv16 = ewb[b, pl.ds(q * 16, 16)]
                for u in range(16):
                    nv = nv16[u]
                    e = q * 16 + u
                    for t in range(8):
                        sl = pl.ds(t * 16, 16)
                        gbuf[e, sl] = gbuf[e, sl] * nv
                return _
            lax.fori_loop(0, CH // 16, scale, None)

            pltpu.make_async_copy(col4.at[wid, j],
                                  idxb.at[pl.ds(NBUF + b, 1)],
                                  csem[b]).wait()
            pltpu.async_copy(gbuf, acc.at[idxb.at[NBUF + b]], ssem[b],
                             add=True)

            @pl.when(j + 1 < nch)
            def _():
                pltpu.make_async_copy(
                    row4.at[wid, j], idxb.at[pl.ds(b1, 1)], rsem[b1]).wait()
                pltpu.async_copy(src_hbm.at[idxb.at[b1]], gb[b1], gsem[b1])

            @pl.when(j >= NBUF - 2)
            def _():
                pltpu.make_async_copy(
                    gb[b2], acc.at[idxb.at[NBUF + b2]], ssem[b2]).wait()

            @pl.when(j + 2 < nch)
            def _():
                jn = j + 2
                pltpu.async_copy(row4.at[wid, jn], idxb.at[pl.ds(b2, 1)],
                                 rsem[b2])
                pltpu.async_copy(col4.at[wid, jn],
                                 idxb.at[pl.ds(NBUF + b2, 1)], csem[b2])
                pltpu.async_copy(ew4.at[wid, jn], ewb.at[pl.ds(b2, 1)],
                                 esem[b2])

    for k in range(nch - (NBUF - 2), nch):
        bl = k % NBUF
        pltpu.make_async_copy(gb[bl], acc.at[idxb.at[NBUF + bl]],
                              ssem[bl]).wait()
    plsc.subcore_barrier()

    wdesc = [None, None]
    for q in range(nw):
        b = q % 2
        r0 = sid * per_sub + q * wchunk
        if wdesc[b] is not None:
            wdesc[b].wait()
        pltpu.async_copy(acc.at[pl.ds(r0, wchunk)],
                         gb[b].at[pl.ds(0, wchunk)], gsem[b]).wait()
        wdesc[b] = pltpu.async_copy(
            gb[b].at[pl.ds(0, wchunk)],
            out_hbm.at[cid, pl.ds(r0, wchunk)], ssem[b])
    for b in range(2):
        if wdesc[b] is not None:
            wdesc[b].wait()
    plsc.subcore_barrier()


def _sc_scratch(d):
    return ([pltpu.VMEM((CH, d), F32)] * NBUF
            + [pltpu.VMEM((2 * NBUF, CH), jnp.int32)]
            + [pltpu.VMEM((NBUF, CH), F32)]
            + [pltpu.SemaphoreType.DMA] * (5 * NBUF))


def _sc_scatter_xh(row4, col4, ew4, xd, hd, n_pad):
    nch = row4.shape[1]
    d = xd.shape[1]
    mesh = plsc.VectorSubcoreMesh(core_axis_name="c", subcore_axis_name="s")

    @functools.partial(
        pl.kernel,
        mesh=mesh,
        out_type=[
            jax.ShapeDtypeStruct((NC, n_pad, d), F32),
            jax.ShapeDtypeStruct((NC, n_pad, d), F32),
        ],
        scratch_types=_sc_scratch(d) + [
            pltpu.VMEM_SHARED((n_pad, d), F32),
        ],
    )
    def k(row_hbm, col_hbm, ew_hbm, x_hbm, h_hbm, sx_out, sh_out, *sc):
        cid = lax.axis_index("c")
        sid = lax.axis_index("s")
        wid = cid * NS + sid

        nb = NBUF
        gb = sc[0:nb]
        idxb, ewb = sc[nb], sc[nb + 1]
        base = nb + 2
        gsem, rsem, csem, esem, ssem = (
            sc[base:base + nb], sc[base + nb:base + 2 * nb],
            sc[base + 2 * nb:base + 3 * nb],
            sc[base + 3 * nb:base + 4 * nb],
            sc[base + 4 * nb:base + 5 * nb])
        acc = sc[base + 5 * nb]
        _scatter_phase(x_hbm, sx_out, acc, row_hbm, col_hbm, ew_hbm, wid,
                       gb, idxb, ewb, gsem, rsem, csem, esem, ssem,
                       cid, sid, nch, n_pad)
        _scatter_phase(h_hbm, sh_out, acc, row_hbm, col_hbm, ew_hbm, wid,
                       gb, idxb, ewb, gsem, rsem, csem, esem, ssem,
                       cid, sid, nch, n_pad)

    return k(row4, col4, ew4, xd, hd)


def _sc_scatter1(row4, col4, ew4, src, n_pad):
    nch = row4.shape[1]
    d = src.shape[1]
    mesh = plsc.VectorSubcoreMesh(core_axis_name="c", subcore_axis_name="s")

    @functools.partial(
        pl.kernel,
        mesh=mesh,
        out_type=jax.ShapeDtypeStruct((NC, n_pad, d), F32),
        scratch_types=_sc_scratch(d) + [
            pltpu.VMEM_SHARED((n_pad, d), F32),
        ],
    )
    def k(row_hbm, col_hbm, ew_hbm, src_hbm, s_out, *sc):
        cid = lax.axis_index("c")
        sid = lax.axis_index("s")
        wid = cid * NS + sid

        nb = NBUF
        gb = sc[0:nb]
        idxb, ewb = sc[nb], sc[nb + 1]
        base = nb + 2
        gsem, rsem, csem, esem, ssem = (
            sc[base:base + nb], sc[base + nb:base + 2 * nb],
            sc[base + 2 * nb:base + 3 * nb],
            sc[base + 3 * nb:base + 4 * nb],
            sc[base + 4 * nb:base + 5 * nb])
        acc = sc[base + 5 * nb]
        _scatter_phase(src_hbm, s_out, acc, row_hbm, col_hbm, ew_hbm, wid,
                       gb, idxb, ewb, gsem, rsem, csem, esem, ssem,
                       cid, sid, nch, n_pad)

    return k(row4, col4, ew4, src)


def _tc_prescale(deg0, deg1, x, h, blk):
    n, d = x.shape

    def body(d0, d1, x_r, h_r, dis_o, xd_o, hd_o):
        dg = d0[...] + d1[...]
        pos = dg > 0.0
        dis = jnp.where(pos, lax.rsqrt(jnp.where(pos, dg, 1.0)), 0.0)
        dis_o[...] = dis
        xd_o[...] = x_r[...] * dis
        hd_o[...] = h_r[...] * dis

    return pl.pallas_call(
        body,
        grid=(n // blk,),
        in_specs=[
            pl.BlockSpec((blk, 1), lambda i: (i, 0)),
            pl.BlockSpec((blk, 1), lambda i: (i, 0)),
            pl.BlockSpec((blk, d), lambda i: (i, 0)),
            pl.BlockSpec((blk, d), lambda i: (i, 0)),
        ],
        out_specs=[
            pl.BlockSpec((blk, 1), lambda i: (i, 0)),
            pl.BlockSpec((blk, d), lambda i: (i, 0)),
            pl.BlockSpec((blk, d), lambda i: (i, 0)),
        ],
        out_shape=[
            jax.ShapeDtypeStruct((n, 1), F32),
            jax.ShapeDtypeStruct((n, d), F32),
            jax.ShapeDtypeStruct((n, d), F32),
        ],
    )(deg0, deg1, x, h)


def _tc_gates(x, h, sxp, shp, dis, wzr, bzr, wp, bp, blk):
    n, d = x.shape

    def body(x_r, h_r, sx_r, sh_r, dis_r, wzr_r, bzr_r, wp_r, bp_r,
             z_o, hr_o, hrd_o, p_o):
        nd = -dis_r[...]
        sx = (sx_r[0] + sx_r[1]) * nd
        sh = (sh_r[0] + sh_r[1]) * nd
        u = jnp.concatenate([x_r[...], sx, h_r[...], sh], axis=1)
        g = jnp.dot(u, wzr_r[...], preferred_element_type=F32,
                    precision=lax.Precision.HIGHEST) + bzr_r[...]
        zr = jax.nn.sigmoid(g)
        z_o[...] = zr[:, :d]
        r = zr[:, d:]
        p_o[...] = jnp.dot(u[:, :2 * d], wp_r[...], preferred_element_type=F32,
                           precision=lax.Precision.HIGHEST) + bp_r[...]
        hr = h_r[...] * r
        hr_o[...] = hr
        hrd_o[...] = hr * dis_r[...]

    return pl.pallas_call(
        body,
        grid=(n // blk,),
        in_specs=[
            pl.BlockSpec((blk, d), lambda i: (i, 0)),
            pl.BlockSpec((blk, d), lambda i: (i, 0)),
            pl.BlockSpec((NC, blk, d), lambda i: (0, i, 0)),
            pl.BlockSpec((NC, blk, d), lambda i: (0, i, 0)),
            pl.BlockSpec((blk, 1), lambda i: (i, 0)),
            pl.BlockSpec((4 * d, 2 * d), lambda i: (0, 0)),
            pl.BlockSpec((1, 2 * d), lambda i: (0, 0)),
            pl.BlockSpec((2 * d, d), lambda i: (0, 0)),
            pl.BlockSpec((1, d), lambda i: (0, 0)),
        ],
        out_specs=[
            pl.BlockSpec((blk, d), lambda i: (i, 0)),
            pl.BlockSpec((blk, d), lambda i: (i, 0)),
            pl.BlockSpec((blk, d), lambda i: (i, 0)),
            pl.BlockSpec((blk, d), lambda i: (i, 0)),
        ],
        out_shape=[
            jax.ShapeDtypeStruct((n, d), F32),
            jax.ShapeDtypeStruct((n, d), F32),
            jax.ShapeDtypeStruct((n, d), F32),
            jax.ShapeDtypeStruct((n, d), F32),
        ],
    )(x, h, sxp, shp, dis, wzr, bzr, wp, bp)


def _tc_final(z, hr, p, h, shrp, dis, whh, wenc, benc, blk):
    n, d = z.shape

    def body(z_r, hr_r, p_r, h_r, shr_r, dis_r, whh_r, wenc_r, benc_r,
             h0_o, pool_o, acc):
        i = pl.program_id(0)
        shr = (shr_r[0] + shr_r[1]) * (-dis_r[...])
        v = jnp.concatenate([hr_r[...], shr], axis=1)
        ht = jnp.tanh(p_r[...] + jnp.dot(v, whh_r[...],
                                         preferred_element_type=F32,
                                         precision=lax.Precision.HIGHEST))
        zg = z_r[...]
        h0 = zg * h_r[...] + (1.0 - zg) * ht
        h0_o[...] = h0
        y = jnp.dot(jax.nn.relu(h0), wenc_r[...], preferred_element_type=F32,
                    precision=lax.Precision.HIGHEST) + benc_r[...]
        y = jax.nn.relu(y)

        @pl.when(i == 0)
        def _():
            acc[...] = jnp.zeros_like(acc)

        acc[...] += jnp.sum(y, axis=0, keepdims=True)

        @pl.when(i == pl.num_programs(0) - 1)
        def _():
            pool_o[...] = jax.nn.sigmoid(acc[...] / n)

    return pl.pallas_call(
        body,
        grid=(n // blk,),
        in_specs=[
            pl.BlockSpec((blk, d), lambda i: (i, 0)),
            pl.BlockSpec((blk, d), lambda i: (i, 0)),
            pl.BlockSpec((blk, d), lambda i: (i, 0)),
            pl.BlockSpec((blk, d), lambda i: (i, 0)),
            pl.BlockSpec((NC, blk, d), lambda i: (0, i, 0)),
            pl.BlockSpec((blk, 1), lambda i: (i, 0)),
            pl.BlockSpec((2 * d, d), lambda i: (0, 0)),
            pl.BlockSpec((d, d), lambda i: (0, 0)),
            pl.BlockSpec((1, d), lambda i: (0, 0)),
        ],
        out_specs=[
            pl.BlockSpec((blk, d), lambda i: (i, 0)),
            pl.BlockSpec((1, d), lambda i: (0, 0)),
        ],
        out_shape=[
            jax.ShapeDtypeStruct((n, d), F32),
            jax.ShapeDtypeStruct((1, d), F32),
        ],
        scratch_shapes=[pltpu.VMEM((1, d), F32)],
    )(z, hr, p, h, shrp, dis, whh, wenc, benc)


def kernel(x, edge_index, edge_weight, h, Wxz, bxz, Whz, bhz, Wxr, bxr,
           Whr, bhr, Wxh, bxh, Whh, bhh, Wenc, benc, Wdisc, bdisc):
    n, d = x.shape
    e = edge_index.shape[1]
    del Wdisc, bdisc

    epw = -(-e // NW)
    nch = -(-(-(-epw // CH)) // NBUF) * NBUF
    e_pad = NW * nch * CH
    n_pad = -(-n // (16 * NS)) * (16 * NS)

    row = edge_index[0]
    col = edge_index[1]
    pad = e_pad - e
    pad_idx = (jnp.arange(pad, dtype=jnp.int32) * 37) % n
    row4 = jnp.concatenate([row, pad_idx]).reshape(NW, nch, 1, CH)
    col4 = jnp.concatenate([col, pad_idx]).reshape(NW, nch, 1, CH)
    ew4 = jnp.pad(edge_weight, (0, pad)).reshape(NW, nch, 1, CH)

    chd = 128
    nchd = -(-epw // chd)
    pad_d = NW * nchd * chd - e
    pad_idx_d = (jnp.arange(pad_d, dtype=jnp.int32) * 37) % n
    rowd = jnp.concatenate([row, pad_idx_d]).reshape(NW, nchd, chd)
    ewd = jnp.pad(edge_weight, (0, pad_d)).reshape(NW, nchd, chd)

    deg0, deg1 = _sc_deg(rowd, ewd, n_pad)
    blk = 1000
    dis, xd, hd = _tc_prescale(deg0[:n].reshape(n, 1), deg1[:n].reshape(n, 1),
                               x, h, blk)

    sxp, shp = _sc_scatter_xh(row4, col4, ew4, xd, hd, n_pad)

    wz = jnp.concatenate([Wxz[0], Wxz[1], Whz[0], Whz[1]], axis=0)
    wr = jnp.concatenate([Wxr[0], Wxr[1], Whr[0], Whr[1]], axis=0)
    wzr = jnp.concatenate([wz, wr], axis=1)
    bzr = jnp.concatenate([bxz + bhz, bxr + bhr]).reshape(1, 2 * d)
    wp = jnp.concatenate([Wxh[0], Wxh[1]], axis=0)
    bp = (bxh + bhh).reshape(1, d)

    z, hr, hrd, p = _tc_gates(x, h, sxp, shp, dis, wzr, bzr, wp, bp, blk)

    shrp = _sc_scatter1(row4, col4, ew4, hrd, n_pad)

    whh = jnp.concatenate([Whh[0], Whh[1]], axis=0)
    h0, pool = _tc_final(z, hr, p, h, shrp, dis, whh, Wenc,
                         benc.reshape(1, -1), blk)
    return (pool.reshape(-1), h0)
